# baseline (speedup 1.0000x reference)
import functools

import jax
import jax.numpy as jnp
from jax import lax
from jax.experimental import pallas as pl
from jax.experimental.pallas import tpu as pltpu
from jax.experimental.pallas import tpu_sc as plsc

_L = 16


def _bcast_lane(vec, lane):
    idx = jnp.full((_L, 1), lane, jnp.int32)
    dnums = lax.GatherDimensionNumbers(
        offset_dims=(), collapsed_slice_dims=(0,), start_index_map=(0,))
    return lax.gather(vec, idx, dnums, slice_sizes=(1,),
                      mode=lax.GatherScatterMode.PROMISE_IN_BOUNDS)


def _sc_body(b1c_hbm, w2ct_hbm, b2c_hbm, wdt_hbm, bd_hbm, out_hbm,
             b1c_v, w2ct_v, b2c_v, wdt_v, bd_v, out_v):
    n, out_dim = out_v.shape
    hid = b1c_v.shape[0]
    lat = b2c_v.shape[0]
    wid = lax.axis_index("s") * 2 + lax.axis_index("c")

    @pl.when(wid == 0)
    def _():
        pltpu.sync_copy(b1c_hbm, b1c_v)
        pltpu.sync_copy(w2ct_hbm, w2ct_v)
        pltpu.sync_copy(b2c_hbm, b2c_v)
        pltpu.sync_copy(wdt_hbm, wdt_v)
        pltpu.sync_copy(bd_hbm, bd_v)

        bd_vec = bd_v[...]
        u = [jnp.maximum(b1c_v[pl.ds(k * _L, _L)], 0.0)
             for k in range(hid // _L)]

        c = [b2c_v[pl.ds(k * _L, _L)] for k in range(lat // _L)]
        for j in range(hid):
            uj = _bcast_lane(u[j // _L], j % _L)
            for k in range(lat // _L):
                c[k] = c[k] + uj * w2ct_v[j, pl.ds(k * _L, _L)]

        r = bd_vec
        for j in range(lat):
            cj = _bcast_lane(c[j // _L], j % _L)
            r = r + cj * wdt_v[j, :]

        for i in range(n):
            out_v[i, :] = bd_vec
        out_v[n - 2, :] = r
        pltpu.sync_copy(out_v, out_hbm)


def kernel(z, num_node, edge_index, W1s, b1s, W2s, b2s, W1c, b1c, W2c, b2c, Wd, bd):
    n = edge_index.shape[1] + 1
    hid = b1c.shape[0]
    lat = W2c.shape[0]
    out_dim = Wd.shape[0]

    sc_kernel = pl.kernel(
        _sc_body,
        out_type=jax.ShapeDtypeStruct((n, out_dim), jnp.float32),
        mesh=plsc.VectorSubcoreMesh(core_axis_name="c", subcore_axis_name="s"),
        scratch_types=[
            pltpu.VMEM((hid,), jnp.float32),
            pltpu.VMEM((hid, lat), jnp.float32),
            pltpu.VMEM((lat,), jnp.float32),
            pltpu.VMEM((lat, out_dim), jnp.float32),
            pltpu.VMEM((out_dim,), jnp.float32),
            pltpu.VMEM((n, out_dim), jnp.float32),
        ],
    )
    return sc_kernel(b1c, W2c.T, b2c, Wd.T, bd)
